# Initial kernel scaffold; baseline (speedup 1.0000x reference)
import jax
import jax.numpy as jnp
from jax.experimental import pallas as pl


def kernel(edge_index, adj_values, feature, W1, W2):
    raise NotImplementedError("write your pallas kernel here")


# speedup vs baseline: 2.4257x; 2.4257x over previous
import functools

import jax
import jax.numpy as jnp
from jax import lax
from jax.experimental import pallas as pl
from jax.experimental.pallas import tpu as pltpu
from jax.experimental.pallas import tpu_sc as plsc

N = 10000
E = 320000
D = 128

NC = 2
NS = 16
NW = NC * NS

K = 128
CH = 80
EPW = K * CH
EPAD = EPW * NW
NPAD = 10240
OUT_PT = NPAD // NS


def _spmm_sc_body(col_hbm, row_hbm, val_hbm, x_hbm, out_hbm,
                  col_v, row_v, val_v, rows_v, acc, sem):
    c = lax.axis_index("c")
    s = lax.axis_index("s")
    wid = c * NS + s

    zero = jnp.zeros((16,), jnp.float32)

    def _zero_row(k, _):
        for j in range(8):
            rows_v[k, pl.ds(j * 16, 16)] = zero
        return 0

    lax.fori_loop(0, K, _zero_row, 0)
    n_zero_blocks = NPAD // NS // K
    for b in range(n_zero_blocks):
        pltpu.sync_copy(rows_v, acc.at[pl.ds(s * (NPAD // NS) + b * K, K)])
    plsc.subcore_barrier()

    def chunk_body(i, _):
        base = wid * EPW + i * K
        pltpu.sync_copy(col_hbm.at[pl.ds(base, K)], col_v)
        pltpu.sync_copy(row_hbm.at[pl.ds(base, K)], row_v)
        pltpu.sync_copy(val_hbm.at[pl.ds(base, K)], val_v)
        pltpu.async_copy(x_hbm.at[col_v], rows_v, sem).wait()

        def group_body(g, _):
            vv = val_v[pl.ds(g * 16, 16)]
            for i in range(16):
                v = jnp.full((16,), vv[i])
                k = g * 16 + i
                for j in range(8):
                    sl = pl.ds(j * 16, 16)
                    rows_v[k, sl] = rows_v[k, sl] * v
            return 0

        lax.fori_loop(0, K // 16, group_body, 0)
        pltpu.sync_copy(rows_v, acc.at[row_v], add=True)
        return 0

    lax.fori_loop(0, CH, chunk_body, 0)
    plsc.subcore_barrier()

    pltpu.sync_copy(acc.at[pl.ds(s * OUT_PT, OUT_PT)],
                    out_hbm.at[c, pl.ds(s * OUT_PT, OUT_PT)])


_spmm_sc = functools.partial(
    pl.kernel,
    out_type=jax.ShapeDtypeStruct((NC, NPAD, D), jnp.float32),
    mesh=plsc.VectorSubcoreMesh(core_axis_name="c", subcore_axis_name="s"),
    scratch_types=[
        pltpu.VMEM((K,), jnp.int32),
        pltpu.VMEM((K,), jnp.int32),
        pltpu.VMEM((K,), jnp.float32),
        pltpu.VMEM((K, D), jnp.float32),
        pltpu.VMEM_SHARED((NPAD, D), jnp.float32),
        pltpu.SemaphoreType.DMA,
    ],
)(_spmm_sc_body)


_MM_BLK = 1000


def _mm_body(x_ref, w_ref, o_ref):
    o_ref[...] = jnp.dot(x_ref[...], w_ref[...],
                         preferred_element_type=jnp.float32)


def _matmul_tc(x, w):
    return pl.pallas_call(
        _mm_body,
        grid=(N // _MM_BLK,),
        in_specs=[
            pl.BlockSpec((_MM_BLK, D), lambda i: (i, 0)),
            pl.BlockSpec((D, D), lambda i: (0, 0)),
        ],
        out_specs=pl.BlockSpec((_MM_BLK, D), lambda i: (i, 0)),
        out_shape=jax.ShapeDtypeStruct((N, D), jnp.float32),
    )(x, w)


def _combine_relu_mm_body(p_ref, w_ref, o_ref):
    h = jnp.maximum(p_ref[0] + p_ref[1], 0.0)
    o_ref[...] = jnp.dot(h, w_ref[...], preferred_element_type=jnp.float32)


def _combine_relu_mm_tc(p, w):
    return pl.pallas_call(
        _combine_relu_mm_body,
        grid=(N // _MM_BLK,),
        in_specs=[
            pl.BlockSpec((NC, _MM_BLK, D), lambda i: (0, i, 0)),
            pl.BlockSpec((D, D), lambda i: (0, 0)),
        ],
        out_specs=pl.BlockSpec((_MM_BLK, D), lambda i: (i, 0)),
        out_shape=jax.ShapeDtypeStruct((N, D), jnp.float32),
    )(p, w)


def _combine_body(p_ref, o_ref):
    o_ref[...] = p_ref[0] + p_ref[1]


def _combine_tc(p):
    return pl.pallas_call(
        _combine_body,
        grid=(N // _MM_BLK,),
        in_specs=[pl.BlockSpec((NC, _MM_BLK, D), lambda i: (0, i, 0))],
        out_specs=pl.BlockSpec((_MM_BLK, D), lambda i: (i, 0)),
        out_shape=jax.ShapeDtypeStruct((N, D), jnp.float32),
    )(p)


def kernel(edge_index, adj_values, feature, W1, W2):
    row = edge_index[0]
    col = edge_index[1]
    pad = EPAD - E
    colp = jnp.concatenate([col, jnp.zeros((pad,), jnp.int32)])
    rowp = jnp.concatenate([row, jnp.full((pad,), N, jnp.int32)])
    valp = jnp.concatenate([adj_values, jnp.zeros((pad,), jnp.float32)])

    x1 = _matmul_tc(feature, W1)
    p1 = _spmm_sc(colp, rowp, valp, x1)
    h = _combine_relu_mm_tc(p1, W2)
    p2 = _spmm_sc(colp, rowp, valp, h)
    return _combine_tc(p2)
